# baseline (speedup 1.0000x reference)
import functools

import jax
import jax.numpy as jnp
from jax import lax
from jax.experimental import pallas as pl
from jax.experimental.pallas import tpu as pltpu
from jax.experimental.pallas import tpu_sc as plsc

_NN = 10000
_NE = 320000
_NG = 64
_D_IN = 128
_D_HID = 256
_D_OUT = 128

_NC = 2
_NS = 16
_NW = _NC * _NS
_CHUNK = 128

_NP = 10240
_RPT = _NP // _NS
_GRP = 8
_NGRP = 10
_NCHUNK = _GRP * _NGRP
_EPW = _NCHUNK * _CHUNK
_NEP = _EPW * _NW

_C2 = 32
_K2 = 4
_NGRP2 = _EPW // (8 * _C2)

_R = 256
_NBLK = _NP // _R


def _agg_body(table, srci, dsti, acc_out,
              sidx, didx, r0, r1, r2, r3, zblk, acc_sh, sem):
    c = lax.axis_index("c")
    s = lax.axis_index("s")
    wid = c * _NS + s
    bufs = (r0, r1, r2, r3)

    for i in range(8):
        for j in range(8):
            zblk[i, pl.ds(j * 16, 16)] = jnp.zeros((16,), jnp.float32)

    @pl.loop(0, _RPT // 8)
    def zloop(i):
        pltpu.sync_copy(zblk, acc_sh.at[pl.ds(s * _RPT + i * 8, 8)])

    plsc.subcore_barrier()

    @pl.loop(0, _NGRP2)
    def gloop(g):
        pltpu.sync_copy(srci.at[wid, pl.ds(g * 8, 8)], sidx)
        pltpu.sync_copy(dsti.at[wid, pl.ds(g * 8, 8)], didx)
        d0 = [pltpu.async_copy(table.at[sidx.at[j]], bufs[j], sem)
              for j in range(_K2)]
        for j in range(_K2):
            d0[j].wait()
        d1 = [pltpu.async_copy(table.at[sidx.at[_K2 + j]], bufs[j], sem)
              for j in range(_K2)]
        for j in range(_K2):
            pltpu.sync_copy(bufs[j], acc_sh.at[didx.at[j]], add=True)
        for j in range(_K2):
            d1[j].wait()
        for j in range(_K2):
            pltpu.sync_copy(bufs[j], acc_sh.at[didx.at[_K2 + j]], add=True)

    plsc.subcore_barrier()
    pltpu.sync_copy(acc_sh.at[pl.ds(s * _RPT, _RPT)],
                    acc_out.at[pl.ds(c * _NP + s * _RPT, _RPT)])


def _deg_body(ones_hbm, dsti, deg_out, didx, oblk, zblk, deg_sh):
    c = lax.axis_index("c")
    s = lax.axis_index("s")
    wid = c * _NS + s

    for i in range(8):
        for j in range(8):
            zblk[i, pl.ds(j * 16, 16)] = jnp.zeros((16,), jnp.float32)
    pltpu.sync_copy(ones_hbm, oblk)

    @pl.loop(0, _RPT // 8)
    def zloop(i):
        pltpu.sync_copy(zblk, deg_sh.at[pl.ds(s * _RPT + i * 8, 8)])

    plsc.subcore_barrier()

    @pl.loop(0, _NGRP)
    def gloop(g):
        pltpu.sync_copy(dsti.at[wid, pl.ds(g * _GRP, _GRP)], didx)
        for j in range(_GRP):
            pltpu.sync_copy(oblk, deg_sh.at[didx.at[j]], add=True)

    plsc.subcore_barrier()
    pltpu.sync_copy(deg_sh.at[pl.ds(s * _RPT, _RPT)],
                    deg_out.at[pl.ds(c * _NP + s * _RPT, _RPT)])


_sc_mesh = plsc.VectorSubcoreMesh(core_axis_name="c", subcore_axis_name="s")

_agg = functools.partial(
    pl.kernel,
    out_type=jax.ShapeDtypeStruct((_NC * _NP, _D_IN), jnp.float32),
    mesh=_sc_mesh,
    scratch_types=[
        pltpu.VMEM((8, _C2), jnp.int32),
        pltpu.VMEM((8, _C2), jnp.int32),
        pltpu.VMEM((_C2, _D_IN), jnp.float32),
        pltpu.VMEM((_C2, _D_IN), jnp.float32),
        pltpu.VMEM((_C2, _D_IN), jnp.float32),
        pltpu.VMEM((_C2, _D_IN), jnp.float32),
        pltpu.VMEM((8, _D_IN), jnp.float32),
        pltpu.VMEM_SHARED((_NP, _D_IN), jnp.float32),
        pltpu.SemaphoreType.DMA,
    ],
)(_agg_body)

_deg = functools.partial(
    pl.kernel,
    out_type=jax.ShapeDtypeStruct((_NC * _NP, _D_IN), jnp.float32),
    mesh=_sc_mesh,
    scratch_types=[
        pltpu.VMEM((_GRP, _CHUNK), jnp.int32),
        pltpu.VMEM((_CHUNK, _D_IN), jnp.float32),
        pltpu.VMEM((8, _D_IN), jnp.float32),
        pltpu.VMEM_SHARED((_NP, _D_IN), jnp.float32),
    ],
)(_deg_body)


def _dense1_body(acc_ref, deg_ref, x_ref, w1l_ref, b1_ref, w1r_ref,
                 w2l_ref, w2r_ref, p_ref, hr_ref):
    a = acc_ref[0] + acc_ref[1]
    d = deg_ref[0, :, 0:1] + deg_ref[1, :, 0:1]
    agg = a / jnp.maximum(d, 1.0)
    h = (jnp.dot(agg, w1l_ref[...], preferred_element_type=jnp.float32)
         + b1_ref[...]
         + jnp.dot(x_ref[...], w1r_ref[...], preferred_element_type=jnp.float32))
    h = jnp.maximum(h, 0.0)
    p_ref[...] = jnp.dot(h, w2l_ref[...], preferred_element_type=jnp.float32)
    hr_ref[...] = jnp.dot(h, w2r_ref[...], preferred_element_type=jnp.float32)


_dense1 = pl.pallas_call(
    _dense1_body,
    grid=(_NBLK,),
    in_specs=[
        pl.BlockSpec((_NC, _R, _D_IN), lambda i: (0, i, 0)),
        pl.BlockSpec((_NC, _R, _D_IN), lambda i: (0, i, 0)),
        pl.BlockSpec((_R, _D_IN), lambda i: (i, 0)),
        pl.BlockSpec((_D_IN, _D_HID), lambda i: (0, 0)),
        pl.BlockSpec((1, _D_HID), lambda i: (0, 0)),
        pl.BlockSpec((_D_IN, _D_HID), lambda i: (0, 0)),
        pl.BlockSpec((_D_HID, _D_OUT), lambda i: (0, 0)),
        pl.BlockSpec((_D_HID, _D_OUT), lambda i: (0, 0)),
    ],
    out_specs=[
        pl.BlockSpec((_R, _D_OUT), lambda i: (i, 0)),
        pl.BlockSpec((_R, _D_OUT), lambda i: (i, 0)),
    ],
    out_shape=[
        jax.ShapeDtypeStruct((_NP, _D_OUT), jnp.float32),
        jax.ShapeDtypeStruct((_NP, _D_OUT), jnp.float32),
    ],
)


def _dense2_body(acc_ref, deg_ref, hr_ref, b2_ref, batch_ref, out_ref,
                 pool_acc, cnt_acc):
    i = pl.program_id(0)
    a = acc_ref[0] + acc_ref[1]
    d = deg_ref[0, :, 0:1] + deg_ref[1, :, 0:1]
    h2 = a / jnp.maximum(d, 1.0) + b2_ref[...] + hr_ref[...]
    b = batch_ref[0, 0, :]
    gids = lax.broadcasted_iota(jnp.int32, (_R, _NG), 1)
    m = (b[:, None] == gids).astype(jnp.float32)
    pooled = lax.dot_general(m, h2, (((0,), (0,)), ((), ())),
                             preferred_element_type=jnp.float32)
    cnt = lax.dot_general(m, jnp.ones((_R, _D_OUT), jnp.float32),
                          (((0,), (0,)), ((), ())),
                          preferred_element_type=jnp.float32)

    @pl.when(i == 0)
    def _():
        pool_acc[...] = jnp.zeros_like(pool_acc)
        cnt_acc[...] = jnp.zeros_like(cnt_acc)

    pool_acc[...] += pooled
    cnt_acc[...] += cnt

    @pl.when(i == _NBLK - 1)
    def _():
        out_ref[...] = pool_acc[...] / jnp.maximum(cnt_acc[...], 1.0)


_dense2 = pl.pallas_call(
    _dense2_body,
    grid=(_NBLK,),
    in_specs=[
        pl.BlockSpec((_NC, _R, _D_OUT), lambda i: (0, i, 0)),
        pl.BlockSpec((_NC, _R, _D_IN), lambda i: (0, i, 0)),
        pl.BlockSpec((_R, _D_OUT), lambda i: (i, 0)),
        pl.BlockSpec((1, _D_OUT), lambda i: (0, 0)),
        pl.BlockSpec((1, 1, _R), lambda i: (i, 0, 0)),
    ],
    out_specs=pl.BlockSpec((_NG, _D_OUT), lambda i: (0, 0)),
    out_shape=jax.ShapeDtypeStruct((_NG, _D_OUT), jnp.float32),
    scratch_shapes=[
        pltpu.VMEM((_NG, _D_OUT), jnp.float32),
        pltpu.VMEM((_NG, _D_OUT), jnp.float32),
    ],
)


def kernel(x, edge_index, batch, W1_l, b1, W1_r, W2_l, b2, W2_r):
    x = x.astype(jnp.float32)
    src = edge_index[0].astype(jnp.int32)
    dst = edge_index[1].astype(jnp.int32)
    pad_e = _NEP - _NE
    src_f = jnp.concatenate([src, jnp.zeros((pad_e,), jnp.int32)])
    dst_f = jnp.concatenate([dst, jnp.full((pad_e,), _NN, jnp.int32)])
    src_a = src_f.reshape(_NW, _EPW // _C2, _C2)
    dst_a = dst_f.reshape(_NW, _EPW // _C2, _C2)
    dst_p = dst_f.reshape(_NW, _NCHUNK, _CHUNK)
    x_p = jnp.pad(x, ((0, _NP - _NN), (0, 0)))
    ones_blk = jnp.ones((_CHUNK, _D_IN), jnp.float32)
    batch3 = jnp.concatenate(
        [batch.astype(jnp.int32), jnp.full((_NP - _NN,), _NG, jnp.int32)]
    ).reshape(_NBLK, 1, _R)

    acc1 = _agg(x_p, src_a, dst_a).reshape(_NC, _NP, _D_IN)
    deg = _deg(ones_blk, dst_p).reshape(_NC, _NP, _D_IN)
    p, hr = _dense1(acc1, deg, x_p, W1_l, b1.reshape(1, _D_HID), W1_r, W2_l, W2_r)
    acc2 = _agg(p, src_a, dst_a).reshape(_NC, _NP, _D_OUT)
    return _dense2(acc2, deg, hr, b2.reshape(1, _D_OUT), batch3)


# speedup vs baseline: 1.1676x; 1.1676x over previous
import functools

import jax
import jax.numpy as jnp
from jax import lax
from jax.experimental import pallas as pl
from jax.experimental.pallas import tpu as pltpu
from jax.experimental.pallas import tpu_sc as plsc

_NN = 10000
_NE = 320000
_NG = 64
_D_IN = 128
_D_HID = 256
_D_OUT = 128

_NC = 2
_NS = 16
_NW = _NC * _NS
_CHUNK = 128

_NP = 10240
_RPT = _NP // _NS
_GRP = 8
_NGRP = 10
_NCHUNK = _GRP * _NGRP
_EPW = _NCHUNK * _CHUNK
_NEP = _EPW * _NW

_C2 = 32
_K2 = 4
_NGTOT = _NEP // (8 * _C2)
_G0 = 58
_G1 = 22

_R = 256
_NBLK = _NP // _R


def _agg_body(table, srci, dsti, acc_out,
              sidx, didx, r0, r1, r2, r3, zblk, acc_sh, sem):
    c = lax.axis_index("c")
    s = lax.axis_index("s")
    wid = c * _NS + s
    bufs = (r0, r1, r2, r3)

    for i in range(8):
        for j in range(8):
            zblk[i, pl.ds(j * 16, 16)] = jnp.zeros((16,), jnp.float32)

    @pl.loop(0, _RPT // 8)
    def zloop(i):
        pltpu.sync_copy(zblk, acc_sh.at[pl.ds(s * _RPT + i * 8, 8)])

    plsc.subcore_barrier()

    ng = jnp.where(c == 0, _G0, _G1)
    gbase = jnp.where(c == 0, s * _G0, _NS * _G0 + s * _G1)

    @pl.loop(0, ng)
    def gloop(g):
        gi = gbase + g
        pltpu.sync_copy(srci.at[gi], sidx)
        pltpu.sync_copy(dsti.at[gi], didx)
        d0 = [pltpu.async_copy(table.at[sidx.at[j]], bufs[j], sem)
              for j in range(_K2)]
        for j in range(_K2):
            d0[j].wait()
        d1 = [pltpu.async_copy(table.at[sidx.at[_K2 + j]], bufs[j], sem)
              for j in range(_K2)]
        for j in range(_K2):
            pltpu.sync_copy(bufs[j], acc_sh.at[didx.at[j]], add=True)
        for j in range(_K2):
            d1[j].wait()
        for j in range(_K2):
            pltpu.sync_copy(bufs[j], acc_sh.at[didx.at[_K2 + j]], add=True)

    plsc.subcore_barrier()
    pltpu.sync_copy(acc_sh.at[pl.ds(s * _RPT, _RPT)],
                    acc_out.at[pl.ds(c * _NP + s * _RPT, _RPT)])


def _deg_body(ones_hbm, dsti, deg_out, didx, oblk, zblk, deg_sh):
    c = lax.axis_index("c")
    s = lax.axis_index("s")
    wid = c * _NS + s

    for i in range(8):
        for j in range(8):
            zblk[i, pl.ds(j * 16, 16)] = jnp.zeros((16,), jnp.float32)
    pltpu.sync_copy(ones_hbm, oblk)

    @pl.loop(0, _RPT // 8)
    def zloop(i):
        pltpu.sync_copy(zblk, deg_sh.at[pl.ds(s * _RPT + i * 8, 8)])

    plsc.subcore_barrier()

    @pl.loop(0, _NGRP)
    def gloop(g):
        pltpu.sync_copy(dsti.at[wid, pl.ds(g * _GRP, _GRP)], didx)
        for j in range(_GRP):
            pltpu.sync_copy(oblk, deg_sh.at[didx.at[j]], add=True)

    plsc.subcore_barrier()
    pltpu.sync_copy(deg_sh.at[pl.ds(s * _RPT, _RPT)],
                    deg_out.at[pl.ds(c * _NP + s * _RPT, _RPT)])


_sc_mesh = plsc.VectorSubcoreMesh(core_axis_name="c", subcore_axis_name="s")

_agg = functools.partial(
    pl.kernel,
    out_type=jax.ShapeDtypeStruct((_NC * _NP, _D_IN), jnp.float32),
    mesh=_sc_mesh,
    scratch_types=[
        pltpu.VMEM((8, _C2), jnp.int32),
        pltpu.VMEM((8, _C2), jnp.int32),
        pltpu.VMEM((_C2, _D_IN), jnp.float32),
        pltpu.VMEM((_C2, _D_IN), jnp.float32),
        pltpu.VMEM((_C2, _D_IN), jnp.float32),
        pltpu.VMEM((_C2, _D_IN), jnp.float32),
        pltpu.VMEM((8, _D_IN), jnp.float32),
        pltpu.VMEM_SHARED((_NP, _D_IN), jnp.float32),
        pltpu.SemaphoreType.DMA,
    ],
)(_agg_body)

_deg = functools.partial(
    pl.kernel,
    out_type=jax.ShapeDtypeStruct((_NC * _NP, _D_IN), jnp.float32),
    mesh=_sc_mesh,
    scratch_types=[
        pltpu.VMEM((_GRP, _CHUNK), jnp.int32),
        pltpu.VMEM((_CHUNK, _D_IN), jnp.float32),
        pltpu.VMEM((8, _D_IN), jnp.float32),
        pltpu.VMEM_SHARED((_NP, _D_IN), jnp.float32),
    ],
)(_deg_body)


def _dense1_body(acc_ref, deg_ref, x_ref, w1l_ref, b1_ref, w1r_ref,
                 w2l_ref, w2r_ref, p_ref, hr_ref):
    a = acc_ref[0] + acc_ref[1]
    d = deg_ref[0, :, 0:1] + deg_ref[1, :, 0:1]
    agg = a / jnp.maximum(d, 1.0)
    h = (jnp.dot(agg, w1l_ref[...], preferred_element_type=jnp.float32)
         + b1_ref[...]
         + jnp.dot(x_ref[...], w1r_ref[...], preferred_element_type=jnp.float32))
    h = jnp.maximum(h, 0.0)
    p_ref[...] = jnp.dot(h, w2l_ref[...], preferred_element_type=jnp.float32)
    hr_ref[...] = jnp.dot(h, w2r_ref[...], preferred_element_type=jnp.float32)


_dense1 = pl.pallas_call(
    _dense1_body,
    grid=(_NBLK,),
    in_specs=[
        pl.BlockSpec((_NC, _R, _D_IN), lambda i: (0, i, 0)),
        pl.BlockSpec((_NC, _R, _D_IN), lambda i: (0, i, 0)),
        pl.BlockSpec((_R, _D_IN), lambda i: (i, 0)),
        pl.BlockSpec((_D_IN, _D_HID), lambda i: (0, 0)),
        pl.BlockSpec((1, _D_HID), lambda i: (0, 0)),
        pl.BlockSpec((_D_IN, _D_HID), lambda i: (0, 0)),
        pl.BlockSpec((_D_HID, _D_OUT), lambda i: (0, 0)),
        pl.BlockSpec((_D_HID, _D_OUT), lambda i: (0, 0)),
    ],
    out_specs=[
        pl.BlockSpec((_R, _D_OUT), lambda i: (i, 0)),
        pl.BlockSpec((_R, _D_OUT), lambda i: (i, 0)),
    ],
    out_shape=[
        jax.ShapeDtypeStruct((_NP, _D_OUT), jnp.float32),
        jax.ShapeDtypeStruct((_NP, _D_OUT), jnp.float32),
    ],
)


def _dense2_body(acc_ref, deg_ref, hr_ref, b2_ref, batch_ref, out_ref,
                 pool_acc, cnt_acc):
    i = pl.program_id(0)
    a = acc_ref[0] + acc_ref[1]
    d = deg_ref[0, :, 0:1] + deg_ref[1, :, 0:1]
    h2 = a / jnp.maximum(d, 1.0) + b2_ref[...] + hr_ref[...]
    b = batch_ref[0, 0, :]
    gids = lax.broadcasted_iota(jnp.int32, (_R, _NG), 1)
    m = (b[:, None] == gids).astype(jnp.float32)
    pooled = lax.dot_general(m, h2, (((0,), (0,)), ((), ())),
                             preferred_element_type=jnp.float32)
    cnt = lax.dot_general(m, jnp.ones((_R, _D_OUT), jnp.float32),
                          (((0,), (0,)), ((), ())),
                          preferred_element_type=jnp.float32)

    @pl.when(i == 0)
    def _():
        pool_acc[...] = jnp.zeros_like(pool_acc)
        cnt_acc[...] = jnp.zeros_like(cnt_acc)

    pool_acc[...] += pooled
    cnt_acc[...] += cnt

    @pl.when(i == _NBLK - 1)
    def _():
        out_ref[...] = pool_acc[...] / jnp.maximum(cnt_acc[...], 1.0)


_dense2 = pl.pallas_call(
    _dense2_body,
    grid=(_NBLK,),
    in_specs=[
        pl.BlockSpec((_NC, _R, _D_OUT), lambda i: (0, i, 0)),
        pl.BlockSpec((_NC, _R, _D_IN), lambda i: (0, i, 0)),
        pl.BlockSpec((_R, _D_OUT), lambda i: (i, 0)),
        pl.BlockSpec((1, _D_OUT), lambda i: (0, 0)),
        pl.BlockSpec((1, 1, _R), lambda i: (i, 0, 0)),
    ],
    out_specs=pl.BlockSpec((_NG, _D_OUT), lambda i: (0, 0)),
    out_shape=jax.ShapeDtypeStruct((_NG, _D_OUT), jnp.float32),
    scratch_shapes=[
        pltpu.VMEM((_NG, _D_OUT), jnp.float32),
        pltpu.VMEM((_NG, _D_OUT), jnp.float32),
    ],
)


def kernel(x, edge_index, batch, W1_l, b1, W1_r, W2_l, b2, W2_r):
    x = x.astype(jnp.float32)
    src = edge_index[0].astype(jnp.int32)
    dst = edge_index[1].astype(jnp.int32)
    pad_e = _NEP - _NE
    src_f = jnp.concatenate([src, jnp.zeros((pad_e,), jnp.int32)])
    dst_f = jnp.concatenate([dst, jnp.full((pad_e,), _NN, jnp.int32)])
    src_a = src_f.reshape(_NGTOT, 8, _C2)
    dst_a = dst_f.reshape(_NGTOT, 8, _C2)
    dst_p = dst_f.reshape(_NW, _NCHUNK, _CHUNK)
    x_p = jnp.pad(x, ((0, _NP - _NN), (0, 0)))
    ones_blk = jnp.ones((_CHUNK, _D_IN), jnp.float32)
    batch3 = jnp.concatenate(
        [batch.astype(jnp.int32), jnp.full((_NP - _NN,), _NG, jnp.int32)]
    ).reshape(_NBLK, 1, _R)

    acc1 = _agg(x_p, src_a, dst_a).reshape(_NC, _NP, _D_IN)
    deg = _deg(ones_blk, dst_p).reshape(_NC, _NP, _D_IN)
    p, hr = _dense1(acc1, deg, x_p, W1_l, b1.reshape(1, _D_HID), W1_r, W2_l, W2_r)
    acc2 = _agg(p, src_a, dst_a).reshape(_NC, _NP, _D_OUT)
    return _dense2(acc2, deg, hr, b2.reshape(1, _D_OUT), batch3)


# speedup vs baseline: 1.2441x; 1.0655x over previous
import functools

import jax
import jax.numpy as jnp
from jax import lax
from jax.experimental import pallas as pl
from jax.experimental.pallas import tpu as pltpu
from jax.experimental.pallas import tpu_sc as plsc

_NN = 10000
_NE = 320000
_NG = 64
_D_IN = 128
_D_HID = 256
_D_OUT = 128

_NC = 2
_NS = 16
_NW = _NC * _NS
_CHUNK = 128

_NP = 10240
_RPT = _NP // _NS
_GRP = 8
_NGRP = 10
_NCHUNK = _GRP * _NGRP
_EPW = _NCHUNK * _CHUNK
_NEP = _EPW * _NW

_C2 = 32
_K2 = 4
_NGTOT = _NEP // (8 * _C2)
_G0 = 64
_G1 = 16

_R = 256
_NBLK = _NP // _R


def _agg_body(table, srci, dsti, acc_out,
              sidx, didx, r0, r1, r2, r3, zblk, acc_sh, sem):
    c = lax.axis_index("c")
    s = lax.axis_index("s")
    wid = c * _NS + s
    bufs = (r0, r1, r2, r3)

    for i in range(8):
        for j in range(8):
            zblk[i, pl.ds(j * 16, 16)] = jnp.zeros((16,), jnp.float32)

    @pl.loop(0, _RPT // 8)
    def zloop(i):
        pltpu.sync_copy(zblk, acc_sh.at[pl.ds(s * _RPT + i * 8, 8)])

    plsc.subcore_barrier()

    ng = jnp.where(c == 0, _G0, _G1)
    gbase = jnp.where(c == 0, s * _G0, _NS * _G0 + s * _G1)

    @pl.loop(0, ng)
    def gloop(g):
        gi = gbase + g
        pltpu.sync_copy(srci.at[gi], sidx)
        pltpu.sync_copy(dsti.at[gi], didx)
        d0 = [pltpu.async_copy(table.at[sidx.at[j]], bufs[j], sem)
              for j in range(_K2)]
        for j in range(_K2):
            d0[j].wait()
        d1 = [pltpu.async_copy(table.at[sidx.at[_K2 + j]], bufs[j], sem)
              for j in range(_K2)]
        for j in range(_K2):
            pltpu.sync_copy(bufs[j], acc_sh.at[didx.at[j]], add=True)
        for j in range(_K2):
            d1[j].wait()
        for j in range(_K2):
            pltpu.sync_copy(bufs[j], acc_sh.at[didx.at[_K2 + j]], add=True)

    plsc.subcore_barrier()
    pltpu.sync_copy(acc_sh.at[pl.ds(s * _RPT, _RPT)],
                    acc_out.at[pl.ds(c * _NP + s * _RPT, _RPT)])


def _deg_body(ones_hbm, dsti, deg_out, didx, oblk, zblk, deg_sh):
    c = lax.axis_index("c")
    s = lax.axis_index("s")
    wid = c * _NS + s

    for i in range(8):
        for j in range(8):
            zblk[i, pl.ds(j * 16, 16)] = jnp.zeros((16,), jnp.float32)
    pltpu.sync_copy(ones_hbm, oblk)

    @pl.loop(0, _RPT // 8)
    def zloop(i):
        pltpu.sync_copy(zblk, deg_sh.at[pl.ds(s * _RPT + i * 8, 8)])

    plsc.subcore_barrier()

    @pl.loop(0, _NGRP)
    def gloop(g):
        pltpu.sync_copy(dsti.at[wid, pl.ds(g * _GRP, _GRP)], didx)
        for j in range(_GRP):
            pltpu.sync_copy(oblk, deg_sh.at[didx.at[j]], add=True)

    plsc.subcore_barrier()
    pltpu.sync_copy(deg_sh.at[pl.ds(s * _RPT, _RPT)],
                    deg_out.at[pl.ds(c * _NP + s * _RPT, _RPT)])


_sc_mesh = plsc.VectorSubcoreMesh(core_axis_name="c", subcore_axis_name="s")

_agg = functools.partial(
    pl.kernel,
    out_type=jax.ShapeDtypeStruct((_NC * _NP, _D_IN), jnp.float32),
    mesh=_sc_mesh,
    scratch_types=[
        pltpu.VMEM((8, _C2), jnp.int32),
        pltpu.VMEM((8, _C2), jnp.int32),
        pltpu.VMEM((_C2, _D_IN), jnp.float32),
        pltpu.VMEM((_C2, _D_IN), jnp.float32),
        pltpu.VMEM((_C2, _D_IN), jnp.float32),
        pltpu.VMEM((_C2, _D_IN), jnp.float32),
        pltpu.VMEM((8, _D_IN), jnp.float32),
        pltpu.VMEM_SHARED((_NP, _D_IN), jnp.float32),
        pltpu.SemaphoreType.DMA,
    ],
)(_agg_body)

_deg = functools.partial(
    pl.kernel,
    out_type=jax.ShapeDtypeStruct((_NC * _NP, _D_IN), jnp.float32),
    mesh=_sc_mesh,
    scratch_types=[
        pltpu.VMEM((_GRP, _CHUNK), jnp.int32),
        pltpu.VMEM((_CHUNK, _D_IN), jnp.float32),
        pltpu.VMEM((8, _D_IN), jnp.float32),
        pltpu.VMEM_SHARED((_NP, _D_IN), jnp.float32),
    ],
)(_deg_body)


def _dense1_body(acc_ref, deg_ref, x_ref, w1l_ref, b1_ref, w1r_ref,
                 w2l_ref, w2r_ref, p_ref, hr_ref):
    a = acc_ref[0] + acc_ref[1]
    d = deg_ref[0, :, 0:1] + deg_ref[1, :, 0:1]
    agg = a / jnp.maximum(d, 1.0)
    h = (jnp.dot(agg, w1l_ref[...], preferred_element_type=jnp.float32)
         + b1_ref[...]
         + jnp.dot(x_ref[...], w1r_ref[...], preferred_element_type=jnp.float32))
    h = jnp.maximum(h, 0.0)
    p_ref[...] = jnp.dot(h, w2l_ref[...], preferred_element_type=jnp.float32)
    hr_ref[...] = jnp.dot(h, w2r_ref[...], preferred_element_type=jnp.float32)


_dense1 = pl.pallas_call(
    _dense1_body,
    grid=(_NBLK,),
    in_specs=[
        pl.BlockSpec((_NC, _R, _D_IN), lambda i: (0, i, 0)),
        pl.BlockSpec((_NC, _R, _D_IN), lambda i: (0, i, 0)),
        pl.BlockSpec((_R, _D_IN), lambda i: (i, 0)),
        pl.BlockSpec((_D_IN, _D_HID), lambda i: (0, 0)),
        pl.BlockSpec((1, _D_HID), lambda i: (0, 0)),
        pl.BlockSpec((_D_IN, _D_HID), lambda i: (0, 0)),
        pl.BlockSpec((_D_HID, _D_OUT), lambda i: (0, 0)),
        pl.BlockSpec((_D_HID, _D_OUT), lambda i: (0, 0)),
    ],
    out_specs=[
        pl.BlockSpec((_R, _D_OUT), lambda i: (i, 0)),
        pl.BlockSpec((_R, _D_OUT), lambda i: (i, 0)),
    ],
    out_shape=[
        jax.ShapeDtypeStruct((_NP, _D_OUT), jnp.float32),
        jax.ShapeDtypeStruct((_NP, _D_OUT), jnp.float32),
    ],
)


def _dense2_body(acc_ref, deg_ref, hr_ref, b2_ref, batch_ref, out_ref,
                 pool_acc, cnt_acc):
    i = pl.program_id(0)
    a = acc_ref[0] + acc_ref[1]
    d = deg_ref[0, :, 0:1] + deg_ref[1, :, 0:1]
    h2 = a / jnp.maximum(d, 1.0) + b2_ref[...] + hr_ref[...]
    b = batch_ref[0, 0, :]
    gids = lax.broadcasted_iota(jnp.int32, (_R, _NG), 1)
    m = (b[:, None] == gids).astype(jnp.float32)
    pooled = lax.dot_general(m, h2, (((0,), (0,)), ((), ())),
                             preferred_element_type=jnp.float32)
    cnt = lax.dot_general(m, jnp.ones((_R, _D_OUT), jnp.float32),
                          (((0,), (0,)), ((), ())),
                          preferred_element_type=jnp.float32)

    @pl.when(i == 0)
    def _():
        pool_acc[...] = jnp.zeros_like(pool_acc)
        cnt_acc[...] = jnp.zeros_like(cnt_acc)

    pool_acc[...] += pooled
    cnt_acc[...] += cnt

    @pl.when(i == _NBLK - 1)
    def _():
        out_ref[...] = pool_acc[...] / jnp.maximum(cnt_acc[...], 1.0)


_dense2 = pl.pallas_call(
    _dense2_body,
    grid=(_NBLK,),
    in_specs=[
        pl.BlockSpec((_NC, _R, _D_OUT), lambda i: (0, i, 0)),
        pl.BlockSpec((_NC, _R, _D_IN), lambda i: (0, i, 0)),
        pl.BlockSpec((_R, _D_OUT), lambda i: (i, 0)),
        pl.BlockSpec((1, _D_OUT), lambda i: (0, 0)),
        pl.BlockSpec((1, 1, _R), lambda i: (i, 0, 0)),
    ],
    out_specs=pl.BlockSpec((_NG, _D_OUT), lambda i: (0, 0)),
    out_shape=jax.ShapeDtypeStruct((_NG, _D_OUT), jnp.float32),
    scratch_shapes=[
        pltpu.VMEM((_NG, _D_OUT), jnp.float32),
        pltpu.VMEM((_NG, _D_OUT), jnp.float32),
    ],
)


def kernel(x, edge_index, batch, W1_l, b1, W1_r, W2_l, b2, W2_r):
    x = x.astype(jnp.float32)
    src = edge_index[0].astype(jnp.int32)
    dst = edge_index[1].astype(jnp.int32)
    pad_e = _NEP - _NE
    src_f = jnp.concatenate([src, jnp.zeros((pad_e,), jnp.int32)])
    dst_f = jnp.concatenate([dst, jnp.full((pad_e,), _NN, jnp.int32)])
    src_a = src_f.reshape(_NGTOT, 8, _C2)
    dst_a = dst_f.reshape(_NGTOT, 8, _C2)
    dst_p = dst_f.reshape(_NW, _NCHUNK, _CHUNK)
    x_p = jnp.pad(x, ((0, _NP - _NN), (0, 0)))
    ones_blk = jnp.ones((_CHUNK, _D_IN), jnp.float32)
    batch3 = jnp.concatenate(
        [batch.astype(jnp.int32), jnp.full((_NP - _NN,), _NG, jnp.int32)]
    ).reshape(_NBLK, 1, _R)

    acc1 = _agg(x_p, src_a, dst_a).reshape(_NC, _NP, _D_IN)
    deg = _deg(ones_blk, dst_p).reshape(_NC, _NP, _D_IN)
    p, hr = _dense1(acc1, deg, x_p, W1_l, b1.reshape(1, _D_HID), W1_r, W2_l, W2_r)
    acc2 = _agg(p, src_a, dst_a).reshape(_NC, _NP, _D_OUT)
    return _dense2(acc2, deg, hr, b2.reshape(1, _D_OUT), batch3)
